# baseline (speedup 1.0000x reference)
import functools

import jax
import jax.numpy as jnp
from jax import lax
from jax.experimental import pallas as pl
from jax.experimental.pallas import tpu as pltpu
from jax.experimental.pallas import tpu_sc as plsc

B = 16384
UD = 64
ZD = 32
BT = 2048
OH = 128

_NC = 2
_NS = 16
_NW = _NC * _NS
_BPW = B // _NW
_CH = 128
_NCH = _BPW // _CH


def _sc_gather_body(uid_hbm, zid_hbm, up_hbm, zp_hbm, uout_hbm, zout_hbm,
                    uidx_v, utid_v, zidx_v, ztid_v,
                    ua_v, ub_v, za_v, zb_v, usem, zsem):
    wid = lax.axis_index("s") * _NC + lax.axis_index("c")
    base = wid * _BPW
    pltpu.sync_copy(uid_hbm.at[pl.ds(wid * _NCH, _NCH)], uidx_v)
    pltpu.sync_copy(zid_hbm.at[pl.ds(wid * _NCH, _NCH)], zidx_v)
    for k in range(_BPW // 16):
        r, c = k // 8, (k % 8) * 16
        utid_v[r, pl.ds(c, 16)] = uidx_v[r, pl.ds(c, 16)] >> 1
        ztid_v[r, pl.ds(c, 16)] = zidx_v[r, pl.ds(c, 16)] >> 2

    ubufs, zbufs = (ua_v, ub_v), (za_v, zb_v)

    def gather(c, b):
        cu = pltpu.async_copy(up_hbm.at[utid_v.at[c]], ubufs[b], usem)
        cz = pltpu.async_copy(zp_hbm.at[ztid_v.at[c]], zbufs[b], zsem)
        return cu, cz

    cps = [gather(0, 0), None]
    for c in range(_NCH):
        b = c & 1
        for cp in cps[b]:
            cp.wait()
        if c + 1 < _NCH:
            cps[1 - b] = gather(c + 1, 1 - b)
        row0 = base + c * _CH
        pltpu.sync_copy(ubufs[b], uout_hbm.at[pl.ds(row0, _CH)])
        pltpu.sync_copy(zbufs[b], zout_hbm.at[pl.ds(row0, _CH)])


@functools.cache
def _sc_gather():
    return functools.partial(
        pl.kernel,
        mesh=plsc.VectorSubcoreMesh(core_axis_name="c", subcore_axis_name="s"),
        out_type=[jax.ShapeDtypeStruct((B, 128), jnp.float32),
                  jax.ShapeDtypeStruct((B, 128), jnp.float32)],
        scratch_types=[pltpu.VMEM((_NCH, _CH), jnp.int32),
                       pltpu.VMEM((_NCH, _CH), jnp.int32),
                       pltpu.VMEM((_NCH, _CH), jnp.int32),
                       pltpu.VMEM((_NCH, _CH), jnp.int32),
                       pltpu.VMEM((_CH, 128), jnp.float32),
                       pltpu.VMEM((_CH, 128), jnp.float32),
                       pltpu.VMEM((_CH, 128), jnp.float32),
                       pltpu.VMEM((_CH, 128), jnp.float32),
                       pltpu.SemaphoreType.DMA,
                       pltpu.SemaphoreType.DMA],
    )(_sc_gather_body)


def _tc_body(idx_ref, up_ref, zp_ref, s0_ref, s1_ref, zw_ref, ws_ref,
             b1_ref, w2_ref, b2_ref, o_ref):
    g = idx_ref[:, 0:1]
    occ = idx_ref[:, 1:2]
    age = idx_ref[:, 2:3]
    dow = idx_ref[:, 3:4]
    hod = idx_ref[:, 4:5]
    upar = idx_ref[:, 5:6]
    zpar = idx_ref[:, 6:7]
    cross = lax.rem(dow * 24 + hod, 34)
    cols = lax.broadcasted_iota(jnp.int32, (BT, OH), 1)
    oh = ((cols == g)
          | (cols == occ + 2)
          | (cols == age + 24)
          | (cols == dow + 31)
          | (cols == hod + 38)
          | (cols == cross + 62)).astype(jnp.float32)
    up = up_ref[...]
    zp = zp_ref[...]
    pf = (upar == 1).astype(jnp.float32)
    user = (jnp.dot(up, s0_ref[...], preferred_element_type=jnp.float32)
            * (1.0 - pf)
            + jnp.dot(up, s1_ref[...], preferred_element_type=jnp.float32)
            * pf)
    zipc = 0.0
    for k in range(4):
        qf = (zpar == k).astype(jnp.float32)
        zipc = zipc + qf * jnp.dot(zp, zw_ref[k], preferred_element_type=jnp.float32)
    h1 = (user + zipc
          + jnp.dot(oh, ws_ref[...], preferred_element_type=jnp.float32)
          + b1_ref[...])
    h1 = jnp.maximum(h1, 0.0)
    h2 = jnp.dot(h1, w2_ref[...], preferred_element_type=jnp.float32) + b2_ref[...]
    h2 = jnp.maximum(h2, 0.0)
    ssq = jnp.sum(h2 * h2, axis=1, keepdims=True)
    o_ref[...] = h2 * lax.rsqrt(jnp.maximum(ssq, 1e-12))


def _tc_tower(idx_packed, up, zp, s0, s1, zw, ws, b1r, w2, b2r,
              interpret=False):
    return pl.pallas_call(
        _tc_body,
        grid=(B // BT,),
        in_specs=[
            pl.BlockSpec((BT, 8), lambda i: (i, 0)),
            pl.BlockSpec((BT, 128), lambda i: (i, 0)),
            pl.BlockSpec((BT, 128), lambda i: (i, 0)),
            pl.BlockSpec((128, UD), lambda i: (0, 0)),
            pl.BlockSpec((128, UD), lambda i: (0, 0)),
            pl.BlockSpec((4, 128, UD), lambda i: (0, 0, 0)),
            pl.BlockSpec((OH, UD), lambda i: (0, 0)),
            pl.BlockSpec((1, UD), lambda i: (0, 0)),
            pl.BlockSpec((UD, ZD), lambda i: (0, 0)),
            pl.BlockSpec((1, ZD), lambda i: (0, 0)),
        ],
        out_specs=pl.BlockSpec((BT, ZD), lambda i: (i, 0)),
        out_shape=jax.ShapeDtypeStruct((B, ZD), jnp.float32),
        interpret=interpret,
    )(idx_packed, up, zp, s0, s1, zw, ws, b1r, w2, b2r)


def kernel(user_gender, user_id, user_occupation_label, user_zip_code,
           bucketized_user_age, day_of_week, hour_of_day, user_table,
           zip_table, cross_table, W1, b1, W2, b2):
    uid = user_id.astype(jnp.int32)
    zid = user_zip_code.astype(jnp.int32)
    upv = user_table.reshape(B * 0 + user_table.shape[0] // 2, 128)
    zpv = zip_table.reshape(zip_table.shape[0] // 4, 128)
    up, zp = _sc_gather()(uid.reshape(-1, _CH), zid.reshape(-1, _CH),
                          upv, zpv)

    g = user_gender.astype(jnp.int32)
    zeros = jnp.zeros_like(g)
    idx_packed = jnp.stack(
        [g, user_occupation_label.astype(jnp.int32),
         bucketized_user_age.astype(jnp.int32),
         day_of_week.astype(jnp.int32), hour_of_day.astype(jnp.int32),
         uid & 1, zid & 3, zeros], axis=1)

    w1u = W1[2:66]
    w1z = W1[88:120]
    zpad = jnp.zeros((64, 64), jnp.float32)
    s0 = jnp.concatenate([w1u, zpad], axis=0)
    s1 = jnp.concatenate([zpad, w1u], axis=0)
    zw = jnp.stack([
        jnp.zeros((128, 64), jnp.float32).at[32 * k:32 * k + 32].set(w1z)
        for k in range(4)], axis=0)
    c2 = cross_table @ W1[158:174]
    ws = jnp.concatenate(
        [W1[0:2], W1[66:88], W1[120:158], c2,
         jnp.zeros((OH - 97, 64), jnp.float32)], axis=0)
    return _tc_tower(idx_packed, up, zp, s0, s1, zw, ws,
                     b1.reshape(1, 64), W2, b2.reshape(1, 32))


# speedup vs baseline: 1.9052x; 1.9052x over previous
import functools

import jax
import jax.numpy as jnp
from jax import lax
from jax.experimental import pallas as pl
from jax.experimental.pallas import tpu as pltpu
from jax.experimental.pallas import tpu_sc as plsc

B = 16384
UD = 64
ZD = 32
BT = 2048
OH = 128

_NC = 2
_NS = 16
_NW = _NC * _NS
_BPW = B // _NW
_CH = 128
_NCH = _BPW // _CH


def _sc_gather_body(utid_hbm, ztid_hbm, up_hbm, zp_hbm, uout_hbm, zout_hbm,
                    utid_v, ztid_v, ua_v, ub_v, za_v, zb_v, usem, zsem):
    wid = lax.axis_index("s") * _NC + lax.axis_index("c")
    base = wid * _BPW
    pltpu.sync_copy(utid_hbm.at[pl.ds(wid * _NCH, _NCH)], utid_v)
    pltpu.sync_copy(ztid_hbm.at[pl.ds(wid * _NCH, _NCH)], ztid_v)

    ubufs, zbufs = (ua_v, ub_v), (za_v, zb_v)

    def gather(c, b):
        cu = pltpu.async_copy(up_hbm.at[utid_v.at[c]], ubufs[b], usem)
        cz = pltpu.async_copy(zp_hbm.at[ztid_v.at[c]], zbufs[b], zsem)
        return cu, cz

    cps = [gather(0, 0), None]
    for c in range(_NCH):
        b = c & 1
        for cp in cps[b]:
            cp.wait()
        if c + 1 < _NCH:
            cps[1 - b] = gather(c + 1, 1 - b)
        row0 = base + c * _CH
        pltpu.sync_copy(ubufs[b], uout_hbm.at[pl.ds(row0, _CH)])
        pltpu.sync_copy(zbufs[b], zout_hbm.at[pl.ds(row0, _CH)])


@functools.cache
def _sc_gather():
    return functools.partial(
        pl.kernel,
        mesh=plsc.VectorSubcoreMesh(core_axis_name="c", subcore_axis_name="s"),
        out_type=[jax.ShapeDtypeStruct((B, 128), jnp.float32),
                  jax.ShapeDtypeStruct((B, 128), jnp.float32)],
        scratch_types=[pltpu.VMEM((_NCH, _CH), jnp.int32),
                       pltpu.VMEM((_NCH, _CH), jnp.int32),
                       pltpu.VMEM((_CH, 128), jnp.float32),
                       pltpu.VMEM((_CH, 128), jnp.float32),
                       pltpu.VMEM((_CH, 128), jnp.float32),
                       pltpu.VMEM((_CH, 128), jnp.float32),
                       pltpu.SemaphoreType.DMA,
                       pltpu.SemaphoreType.DMA],
    )(_sc_gather_body)


_TB = 4096
_UNB = 123
_USPLIT = (_UNB - 1) * _TB
_ZNB = 7
_ZSPLIT = (_ZNB - 1) * _TB


def _tr_pair_body(a_ref, b_ref, o_ref):
    o_ref[...] = jnp.concatenate(
        [jnp.transpose(a_ref[...]), jnp.transpose(b_ref[...])], axis=1)


def _tr_quad_body(a_ref, b_ref, c_ref, d_ref, o_ref):
    o_ref[...] = jnp.concatenate(
        [jnp.transpose(a_ref[...]), jnp.transpose(b_ref[...]),
         jnp.transpose(c_ref[...]), jnp.transpose(d_ref[...])], axis=1)


def _tr_user(tt, interpret=False):
    last = 1000000 // _TB
    return pl.pallas_call(
        _tr_pair_body,
        grid=(_UNB,),
        in_specs=[
            pl.BlockSpec((64, _TB),
                         lambda i: (0, jnp.where(i == _UNB - 1, last, i))),
            pl.BlockSpec((64, _TB),
                         lambda i: (0, jnp.where(i == _UNB - 1, last,
                                                 i + _UNB - 1))),
        ],
        out_specs=pl.BlockSpec((_TB, 128), lambda i: (i, 0)),
        out_shape=jax.ShapeDtypeStruct((_UNB * _TB, 128), jnp.float32),
        interpret=interpret,
    )(tt, tt)


def _tr_zip(tt, interpret=False):
    last = 100000 // _TB
    specs = [
        pl.BlockSpec((32, _TB),
                     lambda i, k=k: (0, jnp.where(i == _ZNB - 1, last,
                                                  i + k * (_ZNB - 1))))
        for k in range(4)
    ]
    return pl.pallas_call(
        _tr_quad_body,
        grid=(_ZNB,),
        in_specs=specs,
        out_specs=pl.BlockSpec((_TB, 128), lambda i: (i, 0)),
        out_shape=jax.ShapeDtypeStruct((_ZNB * _TB, 128), jnp.float32),
        interpret=interpret,
    )(tt, tt, tt, tt)


def _tc_body(idx_ref, up_ref, zp_ref, s0_ref, s1_ref, zw_ref, ws_ref,
             b1_ref, w2_ref, b2_ref, o_ref):
    g = idx_ref[:, 0:1]
    occ = idx_ref[:, 1:2]
    age = idx_ref[:, 2:3]
    dow = idx_ref[:, 3:4]
    hod = idx_ref[:, 4:5]
    upar = idx_ref[:, 5:6]
    zpar = idx_ref[:, 6:7]
    cross = lax.rem(dow * 24 + hod, 34)
    cols = lax.broadcasted_iota(jnp.int32, (BT, OH), 1)
    oh = ((cols == g)
          | (cols == occ + 2)
          | (cols == age + 24)
          | (cols == dow + 31)
          | (cols == hod + 38)
          | (cols == cross + 62)).astype(jnp.float32)
    up = up_ref[...]
    zp = zp_ref[...]
    pf = (upar == 1).astype(jnp.float32)
    user = (jnp.dot(up, s0_ref[...], preferred_element_type=jnp.float32)
            * (1.0 - pf)
            + jnp.dot(up, s1_ref[...], preferred_element_type=jnp.float32)
            * pf)
    zipc = 0.0
    for k in range(4):
        qf = (zpar == k).astype(jnp.float32)
        zipc = zipc + qf * jnp.dot(zp, zw_ref[k], preferred_element_type=jnp.float32)
    h1 = (user + zipc
          + jnp.dot(oh, ws_ref[...], preferred_element_type=jnp.float32)
          + b1_ref[...])
    h1 = jnp.maximum(h1, 0.0)
    h2 = jnp.dot(h1, w2_ref[...], preferred_element_type=jnp.float32) + b2_ref[...]
    h2 = jnp.maximum(h2, 0.0)
    ssq = jnp.sum(h2 * h2, axis=1, keepdims=True)
    o_ref[...] = h2 * lax.rsqrt(jnp.maximum(ssq, 1e-12))


def _tc_tower(idx_packed, up, zp, s0, s1, zw, ws, b1r, w2, b2r,
              interpret=False):
    return pl.pallas_call(
        _tc_body,
        grid=(B // BT,),
        in_specs=[
            pl.BlockSpec((BT, 8), lambda i: (i, 0)),
            pl.BlockSpec((BT, 128), lambda i: (i, 0)),
            pl.BlockSpec((BT, 128), lambda i: (i, 0)),
            pl.BlockSpec((128, UD), lambda i: (0, 0)),
            pl.BlockSpec((128, UD), lambda i: (0, 0)),
            pl.BlockSpec((4, 128, UD), lambda i: (0, 0, 0)),
            pl.BlockSpec((OH, UD), lambda i: (0, 0)),
            pl.BlockSpec((1, UD), lambda i: (0, 0)),
            pl.BlockSpec((UD, ZD), lambda i: (0, 0)),
            pl.BlockSpec((1, ZD), lambda i: (0, 0)),
        ],
        out_specs=pl.BlockSpec((BT, ZD), lambda i: (i, 0)),
        out_shape=jax.ShapeDtypeStruct((B, ZD), jnp.float32),
        interpret=interpret,
    )(idx_packed, up, zp, s0, s1, zw, ws, b1r, w2, b2r)


def kernel(user_gender, user_id, user_occupation_label, user_zip_code,
           bucketized_user_age, day_of_week, hour_of_day, user_table,
           zip_table, cross_table, W1, b1, W2, b2):
    uid = user_id.astype(jnp.int32)
    zid = user_zip_code.astype(jnp.int32)
    upv = _tr_user(user_table.T)
    zpv = _tr_zip(zip_table.T)
    utid = jnp.where(uid >= _USPLIT, uid - _USPLIT, uid)
    uhalf = ((uid >= _USPLIT) & (uid < 2 * _USPLIT)).astype(jnp.int32)
    ztid = jnp.where(zid < 4 * _ZSPLIT, zid % _ZSPLIT, zid - 3 * _ZSPLIT)
    zq = jnp.where(zid < 4 * _ZSPLIT, zid // _ZSPLIT, 0)
    up, zp = _sc_gather()(utid.reshape(-1, _CH), ztid.reshape(-1, _CH),
                          upv, zpv)

    g = user_gender.astype(jnp.int32)
    zeros = jnp.zeros_like(g)
    idx_packed = jnp.stack(
        [g, user_occupation_label.astype(jnp.int32),
         bucketized_user_age.astype(jnp.int32),
         day_of_week.astype(jnp.int32), hour_of_day.astype(jnp.int32),
         uhalf, zq, zeros], axis=1)

    w1u = W1[2:66]
    w1z = W1[88:120]
    zpad = jnp.zeros((64, 64), jnp.float32)
    s0 = jnp.concatenate([w1u, zpad], axis=0)
    s1 = jnp.concatenate([zpad, w1u], axis=0)
    zw = jnp.stack([
        jnp.zeros((128, 64), jnp.float32).at[32 * k:32 * k + 32].set(w1z)
        for k in range(4)], axis=0)
    c2 = cross_table @ W1[158:174]
    ws = jnp.concatenate(
        [W1[0:2], W1[66:88], W1[120:158], c2,
         jnp.zeros((OH - 97, 64), jnp.float32)], axis=0)
    return _tc_tower(idx_packed, up, zp, s0, s1, zw, ws,
                     b1.reshape(1, 64), W2, b2.reshape(1, 32))


# speedup vs baseline: 2.0741x; 1.0886x over previous
import functools

import jax
import jax.numpy as jnp
from jax import lax
from jax.experimental import pallas as pl
from jax.experimental.pallas import tpu as pltpu
from jax.experimental.pallas import tpu_sc as plsc

B = 16384
UD = 64
ZD = 32
BT = 2048
OH = 128

_NC = 2
_NS = 16
_NW = _NC * _NS
_BPW = B // _NW
_CH = 128
_NCH = _BPW // _CH


def _sc_gather_body(utid_hbm, ztid_hbm, up_hbm, zp_hbm, uout_hbm, zout_hbm,
                    utid_v, ztid_v, ua_v, ub_v, za_v, zb_v, usem, zsem):
    wid = lax.axis_index("s") * _NC + lax.axis_index("c")
    base = wid * _BPW
    pltpu.sync_copy(utid_hbm.at[pl.ds(wid * _NCH, _NCH)], utid_v)
    pltpu.sync_copy(ztid_hbm.at[pl.ds(wid * _NCH, _NCH)], ztid_v)

    ubufs, zbufs = (ua_v, ub_v), (za_v, zb_v)

    def gather(c, b):
        cu = pltpu.async_copy(up_hbm.at[utid_v.at[c]], ubufs[b], usem)
        cz = pltpu.async_copy(zp_hbm.at[ztid_v.at[c]], zbufs[b], zsem)
        return cu, cz

    cps = [gather(0, 0), None]
    for c in range(_NCH):
        b = c & 1
        for cp in cps[b]:
            cp.wait()
        if c + 1 < _NCH:
            cps[1 - b] = gather(c + 1, 1 - b)
        row0 = base + c * _CH
        pltpu.sync_copy(ubufs[b], uout_hbm.at[pl.ds(row0, _CH)])
        pltpu.sync_copy(zbufs[b], zout_hbm.at[pl.ds(row0, _CH)])


@functools.cache
def _sc_gather():
    return functools.partial(
        pl.kernel,
        mesh=plsc.VectorSubcoreMesh(core_axis_name="c", subcore_axis_name="s"),
        out_type=[jax.ShapeDtypeStruct((B, 128), jnp.float32),
                  jax.ShapeDtypeStruct((B, 128), jnp.float32)],
        scratch_types=[pltpu.VMEM((_NCH, _CH), jnp.int32),
                       pltpu.VMEM((_NCH, _CH), jnp.int32),
                       pltpu.VMEM((_CH, 128), jnp.float32),
                       pltpu.VMEM((_CH, 128), jnp.float32),
                       pltpu.VMEM((_CH, 128), jnp.float32),
                       pltpu.VMEM((_CH, 128), jnp.float32),
                       pltpu.SemaphoreType.DMA,
                       pltpu.SemaphoreType.DMA],
    )(_sc_gather_body)


_TB = 8192
_UNB = 62
_USPLIT = (_UNB - 1) * _TB
_ZNB = 4
_ZSPLIT = (_ZNB - 1) * _TB


def _tr_pair_body(a_ref, b_ref, o_ref):
    o_ref[...] = jnp.concatenate(
        [jnp.transpose(a_ref[...]), jnp.transpose(b_ref[...])], axis=1)


def _tr_quad_body(a_ref, b_ref, c_ref, d_ref, o_ref):
    o_ref[...] = jnp.concatenate(
        [jnp.transpose(a_ref[...]), jnp.transpose(b_ref[...]),
         jnp.transpose(c_ref[...]), jnp.transpose(d_ref[...])], axis=1)


def _tr_user(tt, interpret=False):
    last = 1000000 // _TB
    return pl.pallas_call(
        _tr_pair_body,
        grid=(_UNB,),
        in_specs=[
            pl.BlockSpec((64, _TB),
                         lambda i: (0, jnp.where(i == _UNB - 1, last, i))),
            pl.BlockSpec((64, _TB),
                         lambda i: (0, jnp.where(i == _UNB - 1, last,
                                                 i + _UNB - 1))),
        ],
        out_specs=pl.BlockSpec((_TB, 128), lambda i: (i, 0)),
        out_shape=jax.ShapeDtypeStruct((_UNB * _TB, 128), jnp.float32),
        interpret=interpret,
    )(tt, tt)


def _tr_zip(tt, interpret=False):
    last = 100000 // _TB
    specs = [
        pl.BlockSpec((32, _TB),
                     lambda i, k=k: (0, jnp.where(i == _ZNB - 1, last,
                                                  i + k * (_ZNB - 1))))
        for k in range(4)
    ]
    return pl.pallas_call(
        _tr_quad_body,
        grid=(_ZNB,),
        in_specs=specs,
        out_specs=pl.BlockSpec((_TB, 128), lambda i: (i, 0)),
        out_shape=jax.ShapeDtypeStruct((_ZNB * _TB, 128), jnp.float32),
        interpret=interpret,
    )(tt, tt, tt, tt)


def _tc_body(idx_ref, up_ref, zp_ref, s0_ref, s1_ref, zw_ref, ws_ref,
             b1_ref, w2_ref, b2_ref, o_ref):
    g = idx_ref[:, 0:1]
    occ = idx_ref[:, 1:2]
    age = idx_ref[:, 2:3]
    dow = idx_ref[:, 3:4]
    hod = idx_ref[:, 4:5]
    upar = idx_ref[:, 5:6]
    zpar = idx_ref[:, 6:7]
    cross = lax.rem(dow * 24 + hod, 34)
    cols = lax.broadcasted_iota(jnp.int32, (BT, OH), 1)
    oh = ((cols == g)
          | (cols == occ + 2)
          | (cols == age + 24)
          | (cols == dow + 31)
          | (cols == hod + 38)
          | (cols == cross + 62)).astype(jnp.float32)
    up = up_ref[...]
    zp = zp_ref[...]
    pf = (upar == 1).astype(jnp.float32)
    user = (jnp.dot(up, s0_ref[...], preferred_element_type=jnp.float32)
            * (1.0 - pf)
            + jnp.dot(up, s1_ref[...], preferred_element_type=jnp.float32)
            * pf)
    zipc = 0.0
    for k in range(4):
        qf = (zpar == k).astype(jnp.float32)
        zipc = zipc + qf * jnp.dot(zp, zw_ref[k], preferred_element_type=jnp.float32)
    h1 = (user + zipc
          + jnp.dot(oh, ws_ref[...], preferred_element_type=jnp.float32)
          + b1_ref[...])
    h1 = jnp.maximum(h1, 0.0)
    h2 = jnp.dot(h1, w2_ref[...], preferred_element_type=jnp.float32) + b2_ref[...]
    h2 = jnp.maximum(h2, 0.0)
    ssq = jnp.sum(h2 * h2, axis=1, keepdims=True)
    o_ref[...] = h2 * lax.rsqrt(jnp.maximum(ssq, 1e-12))


def _tc_tower(idx_packed, up, zp, s0, s1, zw, ws, b1r, w2, b2r,
              interpret=False):
    return pl.pallas_call(
        _tc_body,
        grid=(B // BT,),
        in_specs=[
            pl.BlockSpec((BT, 8), lambda i: (i, 0)),
            pl.BlockSpec((BT, 128), lambda i: (i, 0)),
            pl.BlockSpec((BT, 128), lambda i: (i, 0)),
            pl.BlockSpec((128, UD), lambda i: (0, 0)),
            pl.BlockSpec((128, UD), lambda i: (0, 0)),
            pl.BlockSpec((4, 128, UD), lambda i: (0, 0, 0)),
            pl.BlockSpec((OH, UD), lambda i: (0, 0)),
            pl.BlockSpec((1, UD), lambda i: (0, 0)),
            pl.BlockSpec((UD, ZD), lambda i: (0, 0)),
            pl.BlockSpec((1, ZD), lambda i: (0, 0)),
        ],
        out_specs=pl.BlockSpec((BT, ZD), lambda i: (i, 0)),
        out_shape=jax.ShapeDtypeStruct((B, ZD), jnp.float32),
        interpret=interpret,
    )(idx_packed, up, zp, s0, s1, zw, ws, b1r, w2, b2r)


def kernel(user_gender, user_id, user_occupation_label, user_zip_code,
           bucketized_user_age, day_of_week, hour_of_day, user_table,
           zip_table, cross_table, W1, b1, W2, b2):
    uid = user_id.astype(jnp.int32)
    zid = user_zip_code.astype(jnp.int32)
    upv = _tr_user(user_table.T)
    zpv = _tr_zip(zip_table.T)
    utid = jnp.where(uid >= _USPLIT, uid - _USPLIT, uid)
    uhalf = ((uid >= _USPLIT) & (uid < 2 * _USPLIT)).astype(jnp.int32)
    ztid = jnp.where(zid < 4 * _ZSPLIT, zid % _ZSPLIT, zid - 3 * _ZSPLIT)
    zq = jnp.where(zid < 4 * _ZSPLIT, zid // _ZSPLIT, 0)
    up, zp = _sc_gather()(utid.reshape(-1, _CH), ztid.reshape(-1, _CH),
                          upv, zpv)

    g = user_gender.astype(jnp.int32)
    zeros = jnp.zeros_like(g)
    idx_packed = jnp.stack(
        [g, user_occupation_label.astype(jnp.int32),
         bucketized_user_age.astype(jnp.int32),
         day_of_week.astype(jnp.int32), hour_of_day.astype(jnp.int32),
         uhalf, zq, zeros], axis=1)

    w1u = W1[2:66]
    w1z = W1[88:120]
    zpad = jnp.zeros((64, 64), jnp.float32)
    s0 = jnp.concatenate([w1u, zpad], axis=0)
    s1 = jnp.concatenate([zpad, w1u], axis=0)
    zw = jnp.stack([
        jnp.zeros((128, 64), jnp.float32).at[32 * k:32 * k + 32].set(w1z)
        for k in range(4)], axis=0)
    c2 = cross_table @ W1[158:174]
    ws = jnp.concatenate(
        [W1[0:2], W1[66:88], W1[120:158], c2,
         jnp.zeros((OH - 97, 64), jnp.float32)], axis=0)
    return _tc_tower(idx_packed, up, zp, s0, s1, zw, ws,
                     b1.reshape(1, 64), W2, b2.reshape(1, 32))
